# baseline (speedup 1.0000x reference)
import functools
import jax
import jax.numpy as jnp
from jax import lax
from jax.experimental import pallas as pl
from jax.experimental.pallas import tpu as pltpu
from jax.experimental.pallas import tpu_sc as plsc

NN = 10000
EE = 320000
DIN = 128
CC = 64
NC, NS, LANES = 2, 16, 16
NW = NC * NS
NT = 320
NPAD = NW * NT
CAP = 12288
SCAN_CH = 2000
GCH = 48
EPW = EE // NW
K2CH = 80

_BN_S = float(1.0 / (1.0 + 1e-5) ** 0.5)

_MESH = dict(core_axis_name="c", subcore_axis_name="s",
             num_cores=NC, num_subcores=NS)
_SC_PARAMS = pltpu.CompilerParams(needs_layout_passes=False)


def _wid():
    return lax.axis_index("s") * NC + lax.axis_index("c")


def _k1_node_tables(x, W011, b011, gamma0, beta0, ec_W1, ec_b1):
    RB = 1000

    def body(x_r, w_r, b_r, g_r, be_r, w1_r, b1_r, a_r, bb_r):
        h = jnp.dot(x_r[...], w_r[...], preferred_element_type=jnp.float32)
        h = h + b_r[...]
        h = h * (g_r[...] * _BN_S) + be_r[...]
        h = jnp.maximum(h, 0.0)
        w1 = w1_r[...]
        b1 = b1_r[...]
        acols = []
        bcols = []
        for k in range(3):
            w1a = w1[k, :CC, :]
            w1b = w1[k, CC:, :]
            acols.append(jnp.dot(h, w1a - w1b,
                                 preferred_element_type=jnp.float32)
                         + b1[k][None, :])
            bcols.append(jnp.dot(h, w1b, preferred_element_type=jnp.float32))
        z = jnp.zeros((RB, CC), jnp.float32)
        a_r[...] = jnp.concatenate(acols + [z], axis=1)
        bb_r[...] = jnp.concatenate(bcols + [z], axis=1)

    return pl.pallas_call(
        body,
        grid=(NN // RB,),
        in_specs=[
            pl.BlockSpec((RB, DIN), lambda i: (i, 0)),
            pl.BlockSpec((DIN, CC), lambda i: (0, 0)),
            pl.BlockSpec((1, CC), lambda i: (0, 0)),
            pl.BlockSpec((1, CC), lambda i: (0, 0)),
            pl.BlockSpec((1, CC), lambda i: (0, 0)),
            pl.BlockSpec((3, 2 * CC, CC), lambda i: (0, 0, 0)),
            pl.BlockSpec((3, CC), lambda i: (0, 0)),
        ],
        out_specs=[
            pl.BlockSpec((RB, 4 * CC), lambda i: (i, 0)),
            pl.BlockSpec((RB, 4 * CC), lambda i: (i, 0)),
        ],
        out_shape=[
            jax.ShapeDtypeStruct((NN, 4 * CC), jnp.float32),
            jax.ShapeDtypeStruct((NN, 4 * CC), jnp.float32),
        ],
    )(x, W011, b011.reshape(1, CC), gamma0.reshape(1, CC),
      beta0.reshape(1, CC), ec_W1, ec_b1)


def _k2_edge_z(adst, bsrc, src, dst):
    @functools.partial(
        pl.kernel, mesh=plsc.VectorSubcoreMesh(**_MESH),
        compiler_params=_SC_PARAMS,
        out_type=jax.ShapeDtypeStruct((EE, 4 * CC), jnp.float32),
        scratch_types=[
            pltpu.VMEM((K2CH,), jnp.int32),
            pltpu.VMEM((K2CH,), jnp.int32),
            pltpu.VMEM((K2CH, 4 * CC), jnp.float32),
            pltpu.VMEM((K2CH, 4 * CC), jnp.float32),
            pltpu.SemaphoreType.DMA,
            pltpu.SemaphoreType.DMA,
        ],
    )
    def k(adst_h, bsrc_h, src_h, dst_h, z_h, d_v, s_v, ga, gb, sem1, sem2):
        w = _wid()

        def chunk(i, carry):
            base = w * EPW + i * K2CH
            pltpu.sync_copy(dst_h.at[pl.ds(base, K2CH)], d_v)
            pltpu.sync_copy(src_h.at[pl.ds(base, K2CH)], s_v)
            cp1 = pltpu.async_copy(adst_h.at[d_v], ga, sem1)
            cp2 = pltpu.async_copy(bsrc_h.at[s_v], gb, sem2)
            cp1.wait()
            cp2.wait()

            def addrow(r, c2):
                for cg in range(12):
                    sl = pl.ds(cg * LANES, LANES)
                    ga[r, sl] = ga[r, sl] + gb[r, sl]
                return c2

            lax.fori_loop(0, K2CH, addrow, jnp.int32(0))
            pltpu.sync_copy(ga, z_h.at[pl.ds(base, K2CH)])
            return carry

        lax.fori_loop(0, EPW // K2CH, chunk, jnp.int32(0))

    return k(adst, bsrc, src, dst)


def _k3_edge_mlp(z, ec_W2, ec_b2):
    EB = 2000

    def body(z_r, w2_r, b2_r, m_r):
        zb = z_r[...]
        w2 = w2_r[...]
        b2 = b2_r[...]
        cols = []
        for k in range(3):
            zk = jnp.maximum(zb[:, k * CC:(k + 1) * CC], 0.0)
            cols.append(jnp.dot(zk, w2[k], preferred_element_type=jnp.float32)
                        + b2[k][None, :])
        cols.append(jnp.zeros((EB, CC), jnp.float32))
        m_r[...] = jnp.concatenate(cols, axis=1)

    return pl.pallas_call(
        body,
        grid=(EE // EB,),
        in_specs=[
            pl.BlockSpec((EB, 4 * CC), lambda i: (i, 0)),
            pl.BlockSpec((3, CC, CC), lambda i: (0, 0, 0)),
            pl.BlockSpec((3, CC), lambda i: (0, 0)),
        ],
        out_specs=pl.BlockSpec((EB, 4 * CC), lambda i: (i, 0)),
        out_shape=jax.ShapeDtypeStruct((EE, 4 * CC), jnp.float32),
    )(z, ec_W2, ec_b2)


def _scan_compact(dst_h, ea_h, aux_h, sc_d, sc_e, sc_a, pk_l, lo,
                  ldst_shift, ea_shift, use_iota_aux):
    def chunk(c, off):
        base = c * SCAN_CH
        pltpu.sync_copy(dst_h.at[pl.ds(base, SCAN_CH)], sc_d)
        pltpu.sync_copy(ea_h.at[pl.ds(base, SCAN_CH)], sc_e)
        if not use_iota_aux:
            pltpu.sync_copy(aux_h.at[pl.ds(base, SCAN_CH)], sc_a)

        def grp(g, off2):
            v = sc_d[pl.ds(g * LANES, LANES)]
            eav = sc_e[pl.ds(g * LANES, LANES)]
            m = (v >= lo) & (v < lo + NT)
            mi = m.astype(jnp.int32)
            cnt = jnp.sum(mi)
            offg = jnp.minimum(off2, CAP - 16)
            pos = offg + plsc.cumsum(mi) - mi
            if use_iota_aux:
                aux = base + g * LANES + lax.iota(jnp.int32, LANES)
            else:
                aux = sc_a[pl.ds(g * LANES, LANES)]
            pk = aux + ((v - lo) << ldst_shift) + ((eav + 2) << ea_shift)
            plsc.store_scatter(pk_l, [pos], pk, mask=m)
            return off2 + cnt

        return lax.fori_loop(0, SCAN_CH // LANES, grp, off)

    off = lax.fori_loop(0, EE // SCAN_CH, chunk, jnp.int32(0))
    return jnp.minimum(off, CAP - 16)


def _k4_segmax(m_in, dst, ea, bn_gamma, bn_beta):
    NEG = jnp.float32(-jnp.inf)
    AUXM = (1 << 19) - 1

    @functools.partial(
        pl.kernel, mesh=plsc.VectorSubcoreMesh(**_MESH),
        compiler_params=_SC_PARAMS,
        out_type=[jax.ShapeDtypeStruct((NPAD, 2 * CC), jnp.float32),
                  jax.ShapeDtypeStruct((NPAD, 2 * CC), jnp.float32)],
        scratch_types=[
            pltpu.VMEM((SCAN_CH,), jnp.int32),
            pltpu.VMEM((SCAN_CH,), jnp.int32),
            pltpu.VMEM((CAP,), jnp.int32),
            pltpu.VMEM((GCH,), jnp.int32),
            pltpu.VMEM((GCH,), jnp.int32),
            pltpu.VMEM((GCH, 4 * CC), jnp.float32),
            pltpu.VMEM((GCH, 4 * CC), jnp.float32),
            pltpu.VMEM((NT, 2 * CC), jnp.float32),
            pltpu.VMEM((NT, 2 * CC), jnp.float32),
            pltpu.VMEM((3, CC), jnp.float32),
            pltpu.VMEM((3, CC), jnp.float32),
            pltpu.SemaphoreType.DMA,
            pltpu.SemaphoreType.DMA,
        ],
    )
    def k(m_h, dst_h, ea_h, g_h, b_h, x12_h, x3_h, sc_d, sc_e, pk_l,
          eidb0, eidb1, mrows0, mrows1, acc01, acc2z, gam, bet, sem0,
          sem1):
        w = _wid()
        lo = w * NT

        ninf = jnp.full((LANES,), NEG)
        zi = jnp.zeros((LANES,), jnp.int32)

        def init_r(r, c2):
            for cg in range(2 * CC // LANES):
                sl = pl.ds(cg * LANES, LANES)
                acc01[r, sl] = ninf
                acc2z[r, sl] = ninf
            return c2

        lax.fori_loop(0, NT, init_r, jnp.int32(0))

        def init_e(r, c2):
            pk_l[pl.ds(r * LANES, LANES)] = zi
            return c2

        lax.fori_loop(0, CAP // LANES, init_e, jnp.int32(0))

        nk = _scan_compact(dst_h, ea_h, None, sc_d, sc_e, None, pk_l, lo,
                           19, 28, True)
        nch = (nk + GCH - 1) // GCH
        eidbs = (eidb0, eidb1)
        mrowss = (mrows0, mrows1)
        sems = (sem0, sem1)
        ninfv = jnp.full((LANES,), NEG)

        def start_gather(j, b):
            kb = jnp.minimum(j * GCH, CAP - GCH)
            for g in range(GCH // LANES):
                pkv = pk_l[pl.ds(kb + g * LANES, LANES)]
                eidbs[b][pl.ds(g * LANES, LANES)] = pkv & AUXM
            return pltpu.async_copy(m_h.at[eidbs[b]], mrowss[b], sems[b])

        def work(j, b):
            kb = jnp.minimum(j * GCH, CAP - GCH)
            mrows = mrowss[b]

            def grp(g, c3):
                pkv = pk_l[pl.ds(kb + g * LANES, LANES)]
                for lane in range(LANES):
                    pk = pkv[lane]
                    ldst = (pk >> 19) & 511
                    ea2 = (pk >> 28) & 7
                    va = kb + g * LANES + lane < nk
                    conds = (va & (ea2 <= 2), va & (ea2 >= 2), va)
                    for kc in range(3):
                        a = acc01 if kc < 2 else acc2z
                        ab = (kc % 2) * CC
                        for cg in range(CC // LANES):
                            sl = pl.ds(ab + cg * LANES, LANES)
                            msl = pl.ds(kc * CC + cg * LANES, LANES)
                            mv = jnp.where(conds[kc], mrows[g * LANES
                                                            + lane, msl],
                                           ninfv)
                            a[ldst, sl] = jnp.maximum(a[ldst, sl], mv)
                return c3

            lax.fori_loop(0, GCH // LANES, grp, jnp.int32(0))

        start_gather(0, 0)
        start_gather(1, 1)
        npair = (nch + 1) // 2

        def pair(p, c2):
            j = p * 2
            pltpu.make_async_copy(m_h.at[eidb0], mrows0, sem0).wait()
            work(j, 0)
            start_gather(j + 2, 0)
            pltpu.make_async_copy(m_h.at[eidb1], mrows1, sem1).wait()
            work(j + 1, 1)
            start_gather(j + 3, 1)
            return c2

        lax.fori_loop(0, npair, pair, jnp.int32(0))
        pltpu.make_async_copy(m_h.at[eidb0], mrows0, sem0).wait()
        pltpu.make_async_copy(m_h.at[eidb1], mrows1, sem1).wait()

        pltpu.sync_copy(g_h, gam)
        pltpu.sync_copy(b_h, bet)
        zf = jnp.zeros((LANES,), jnp.float32)

        def fin_r(r, c2):
            for kc in range(3):
                a = acc01 if kc < 2 else acc2z
                cb = (kc % 2) * CC
                for cg in range(CC // LANES):
                    sl = pl.ds(cb + cg * LANES, LANES)
                    gsl = pl.ds(cg * LANES, LANES)
                    v = a[r, sl]
                    v = jnp.where(v == NEG, 0.0, v)
                    v = jnp.maximum(v * (gam[kc, gsl] * _BN_S)
                                    + bet[kc, gsl], 0.0)
                    a[r, sl] = v
            for cg in range(CC // LANES):
                acc2z[r, pl.ds(CC + cg * LANES, LANES)] = zf
            return c2

        lax.fori_loop(0, NT, fin_r, jnp.int32(0))
        pltpu.sync_copy(acc01, x12_h.at[pl.ds(lo, NT)])
        pltpu.sync_copy(acc2z, x3_h.at[pl.ds(lo, NT)])

    return k(m_in, dst, ea, bn_gamma, bn_beta)


def _k5_rgcn_sums(x12, x3, src, dst, ea):
    AUXM = (1 << 14) - 1

    @functools.partial(
        pl.kernel, mesh=plsc.VectorSubcoreMesh(**_MESH),
        compiler_params=_SC_PARAMS,
        out_type=jax.ShapeDtypeStruct((3, NPAD, 2 * CC), jnp.float32),
        scratch_types=[
            pltpu.VMEM((SCAN_CH,), jnp.int32),
            pltpu.VMEM((SCAN_CH,), jnp.int32),
            pltpu.VMEM((SCAN_CH,), jnp.int32),
            pltpu.VMEM((CAP,), jnp.int32),
            pltpu.VMEM((GCH,), jnp.int32),
            pltpu.VMEM((GCH,), jnp.int32),
            pltpu.VMEM((GCH, 2 * CC), jnp.float32),
            pltpu.VMEM((GCH, 2 * CC), jnp.float32),
            pltpu.VMEM((NT, 2 * CC), jnp.float32),
            pltpu.VMEM((NT, 2 * CC), jnp.float32),
            pltpu.SemaphoreType.DMA,
            pltpu.SemaphoreType.DMA,
        ],
    )
    def k(x12_h, x3_h, src_h, dst_h, ea_h, s_out, sc_d, sc_e, sc_s, pk_l,
          srcb0, srcb1, xrows0, xrows1, accAB, accCc, sem0, sem1):
        w = _wid()
        lo = w * NT

        zi = jnp.zeros((LANES,), jnp.int32)

        def init_e(r, c2):
            pk_l[pl.ds(r * LANES, LANES)] = zi
            return c2

        lax.fori_loop(0, CAP // LANES, init_e, jnp.int32(0))

        nk = _scan_compact(dst_h, ea_h, src_h, sc_d, sc_e, sc_s, pk_l, lo,
                           14, 23, False)
        nch = (nk + GCH - 1) // GCH

        zf = jnp.zeros((LANES,), jnp.float32)
        one0 = jnp.where(lax.iota(jnp.int32, LANES) == 0, 1.0, 0.0
                         ).astype(jnp.float32)

        def zero_acc(both):
            def init_r(r, c2):
                for cg in range(2 * CC // LANES):
                    sl = pl.ds(cg * LANES, LANES)
                    accAB[r, sl] = zf
                    if both:
                        accCc[r, sl] = zf
                return c2

            lax.fori_loop(0, NT, init_r, jnp.int32(0))

        srcbs = (srcb0, srcb1)
        xrowss = (xrows0, xrows1)
        sems = (sem0, sem1)
        zfv = jnp.zeros((LANES,), jnp.float32)
        il = lax.iota(jnp.int32, LANES)

        for ps in range(2):
            zero_acc(ps == 0)
            xh = x12_h if ps == 0 else x3_h

            def start_gather(j, b, xh=xh):
                kb = jnp.minimum(j * GCH, CAP - GCH)
                for g in range(GCH // LANES):
                    pkv = pk_l[pl.ds(kb + g * LANES, LANES)]
                    srcbs[b][pl.ds(g * LANES, LANES)] = pkv & AUXM
                return pltpu.async_copy(xh.at[srcbs[b]], xrowss[b], sems[b])

            def work(j, b, ps=ps):
                kb = jnp.minimum(j * GCH, CAP - GCH)
                xrows = xrowss[b]

                def grp(g, c3):
                    pkv = pk_l[pl.ds(kb + g * LANES, LANES)]
                    for lane in range(LANES):
                        pk = pkv[lane]
                        ldst = (pk >> 14) & 511
                        ea2 = (pk >> 23) & 7
                        va = kb + g * LANES + lane < nk
                        if ps == 0:
                            combos = ((va & (ea2 == 0), accAB, 0, 0),
                                      (va & ((ea2 == 1) | (ea2 == 2)),
                                       accAB, CC, 0),
                                      (va & (ea2 >= 2), accCc, 0, CC))
                        else:
                            combos = ((va & (ea2 == 0), accAB, 0, 0),
                                      (va & (ea2 > 0), accAB, CC, 0))
                        for (cond, a, ab, xb) in combos:
                            for cg in range(CC // LANES):
                                sl = pl.ds(ab + cg * LANES, LANES)
                                xsl = pl.ds(xb + cg * LANES, LANES)
                                xv = jnp.where(cond, xrows[g * LANES + lane,
                                                           xsl], zfv)
                                a[ldst, sl] = a[ldst, sl] + xv
                        if ps == 0:
                            c0f = (va & (ea2 == 0)).astype(jnp.float32)
                            c1f = (va & ((ea2 == 1) | (ea2 == 2))
                                   ).astype(jnp.float32)
                            c2f = (va & (ea2 >= 2)).astype(jnp.float32)
                            caf = va.astype(jnp.float32)
                            cv = jnp.where(il == 0, c0f,
                                           jnp.where(il == 1, c1f,
                                                     jnp.where(il == 2, c2f,
                                                               jnp.where(
                                                                   il == 3,
                                                                   caf,
                                                                   0.0))))
                            csl = pl.ds(CC, LANES)
                            accCc[ldst, csl] = accCc[ldst, csl] + cv
                    return c3

                lax.fori_loop(0, GCH // LANES, grp, jnp.int32(0))

            start_gather(0, 0)
            start_gather(1, 1)
            npair = (nch + 1) // 2

            def pair(p, c2, xh=xh):
                j = p * 2
                pltpu.make_async_copy(xh.at[srcb0], xrows0, sem0).wait()
                work(j, 0)
                start_gather(j + 2, 0)
                pltpu.make_async_copy(xh.at[srcb1], xrows1, sem1).wait()
                work(j + 1, 1)
                start_gather(j + 3, 1)
                return c2

            lax.fori_loop(0, npair, pair, jnp.int32(0))
            pltpu.make_async_copy(xh.at[srcb0], xrows0, sem0).wait()
            pltpu.make_async_copy(xh.at[srcb1], xrows1, sem1).wait()
            if ps == 0:
                pltpu.sync_copy(accAB, s_out.at[0, pl.ds(lo, NT)])
                pltpu.sync_copy(accCc, s_out.at[1, pl.ds(lo, NT)])
            else:
                pltpu.sync_copy(accAB, s_out.at[2, pl.ds(lo, NT)])

    return k(x12, x3, src, dst, ea)


def _k6_combine(x12, x3, s, rg_W, rg_root, rg_bias):
    RB = 1000

    def body(x12_r, x3_r, s_r, w_r, root_r, bias_r, o_r):
        x12b = x12_r[...]
        x3b = x3_r[...]
        sb = s_r[...]
        roots = root_r[...]
        ws = w_r[...]
        bias = bias_r[...]
        xs = (x12b[:, :CC], x12b[:, CC:], x3b[:, :CC])
        out = jnp.zeros((RB, CC), jnp.float32)
        for kc in range(3):
            out = out + jnp.dot(xs[kc], roots[kc],
                                preferred_element_type=jnp.float32)
            out = out + bias[kc][None, :]
        cnt0 = jnp.maximum(sb[1, :, CC:CC + 1], 1.0)
        cnt1 = jnp.maximum(sb[1, :, CC + 1:CC + 2], 1.0)
        cnt2 = jnp.maximum(sb[1, :, CC + 2:CC + 3], 1.0)
        cnt3 = jnp.maximum(sb[1, :, CC + 3:CC + 4]
                           - sb[1, :, CC:CC + 1], 1.0)
        combos = ((sb[0, :, :CC], cnt0, 0, 0),
                  (sb[0, :, CC:], cnt1, 0, 1),
                  (sb[1, :, :CC], cnt2, 1, 1),
                  (sb[2, :, :CC], cnt0, 2, 0),
                  (sb[2, :, CC:], cnt3, 2, 1))
        for (agg, cnt, kc, r) in combos:
            out = out + jnp.dot(agg / cnt, ws[kc, r],
                                preferred_element_type=jnp.float32)
        o_r[...] = out

    return pl.pallas_call(
        body,
        grid=(NN // RB,),
        in_specs=[
            pl.BlockSpec((RB, 2 * CC), lambda i: (i, 0)),
            pl.BlockSpec((RB, 2 * CC), lambda i: (i, 0)),
            pl.BlockSpec((3, RB, 2 * CC), lambda i: (0, i, 0)),
            pl.BlockSpec((3, 2, CC, CC), lambda i: (0, 0, 0, 0)),
            pl.BlockSpec((3, CC, CC), lambda i: (0, 0, 0)),
            pl.BlockSpec((3, CC), lambda i: (0, 0)),
        ],
        out_specs=pl.BlockSpec((RB, CC), lambda i: (i, 0)),
        out_shape=jax.ShapeDtypeStruct((NN, CC), jnp.float32),
    )(x12, x3, s, rg_W, rg_root, rg_bias)


def kernel(x, edge_index, edge_attr, W011, b011, gamma0, beta0, ec_W1,
           ec_b1, ec_W2, ec_b2, bn_gamma, bn_beta, rg_W, rg_root, rg_bias):
    src = edge_index[0].astype(jnp.int32)
    dst = edge_index[1].astype(jnp.int32)
    ea = edge_attr.astype(jnp.int32)

    adst, bsrc = _k1_node_tables(x, W011, b011, gamma0, beta0, ec_W1, ec_b1)
    z = _k2_edge_z(adst, bsrc, src, dst)
    m = _k3_edge_mlp(z, ec_W2, ec_b2)
    x12, x3 = _k4_segmax(m, dst, ea, bn_gamma, bn_beta)
    s = _k5_rgcn_sums(x12, x3, src, dst, ea)
    return _k6_combine(x12, x3, s, rg_W, rg_root, rg_bias)


# speedup vs baseline: 1.0700x; 1.0700x over previous
import functools
import jax
import jax.numpy as jnp
from jax import lax
from jax.experimental import pallas as pl
from jax.experimental.pallas import tpu as pltpu
from jax.experimental.pallas import tpu_sc as plsc

NN = 10000
EE = 320000
DIN = 128
CC = 64
NC, NS, LANES = 2, 16, 16
NW = NC * NS
NT = 320
NPAD = NW * NT
CAP = 12288
SCAN_CH = 2000
GCH = 48
EPW = EE // NW
K2CH = 80

_BN_S = float(1.0 / (1.0 + 1e-5) ** 0.5)

_MESH = dict(core_axis_name="c", subcore_axis_name="s",
             num_cores=NC, num_subcores=NS)
_SC_PARAMS = pltpu.CompilerParams(needs_layout_passes=False)


def _wid():
    return lax.axis_index("s") * NC + lax.axis_index("c")


def _k1_node_tables(x, W011, b011, gamma0, beta0, ec_W1, ec_b1):
    RB = 1000

    def body(x_r, w_r, b_r, g_r, be_r, w1_r, b1_r, a_r, bb_r):
        h = jnp.dot(x_r[...], w_r[...], preferred_element_type=jnp.float32)
        h = h + b_r[...]
        h = h * (g_r[...] * _BN_S) + be_r[...]
        h = jnp.maximum(h, 0.0)
        w1 = w1_r[...]
        b1 = b1_r[...]
        acols = []
        bcols = []
        for k in range(3):
            w1a = w1[k, :CC, :]
            w1b = w1[k, CC:, :]
            acols.append(jnp.dot(h, w1a - w1b,
                                 preferred_element_type=jnp.float32)
                         + b1[k][None, :])
            bcols.append(jnp.dot(h, w1b, preferred_element_type=jnp.float32))
        z = jnp.zeros((RB, CC), jnp.float32)
        a_r[...] = jnp.concatenate(acols + [z], axis=1)
        bb_r[...] = jnp.concatenate(bcols + [z], axis=1)

    return pl.pallas_call(
        body,
        grid=(NN // RB,),
        in_specs=[
            pl.BlockSpec((RB, DIN), lambda i: (i, 0)),
            pl.BlockSpec((DIN, CC), lambda i: (0, 0)),
            pl.BlockSpec((1, CC), lambda i: (0, 0)),
            pl.BlockSpec((1, CC), lambda i: (0, 0)),
            pl.BlockSpec((1, CC), lambda i: (0, 0)),
            pl.BlockSpec((3, 2 * CC, CC), lambda i: (0, 0, 0)),
            pl.BlockSpec((3, CC), lambda i: (0, 0)),
        ],
        out_specs=[
            pl.BlockSpec((RB, 4 * CC), lambda i: (i, 0)),
            pl.BlockSpec((RB, 4 * CC), lambda i: (i, 0)),
        ],
        out_shape=[
            jax.ShapeDtypeStruct((NN, 4 * CC), jnp.float32),
            jax.ShapeDtypeStruct((NN, 4 * CC), jnp.float32),
        ],
    )(x, W011, b011.reshape(1, CC), gamma0.reshape(1, CC),
      beta0.reshape(1, CC), ec_W1, ec_b1)


def _k2_edge_z(adst, bsrc, src, dst):
    @functools.partial(
        pl.kernel, mesh=plsc.VectorSubcoreMesh(**_MESH),
        compiler_params=_SC_PARAMS,
        out_type=jax.ShapeDtypeStruct((EE, 4 * CC), jnp.float32),
        scratch_types=[
            pltpu.VMEM((K2CH,), jnp.int32),
            pltpu.VMEM((K2CH,), jnp.int32),
            pltpu.VMEM((K2CH, 4 * CC), jnp.float32),
            pltpu.VMEM((K2CH, 4 * CC), jnp.float32),
            pltpu.SemaphoreType.DMA,
            pltpu.SemaphoreType.DMA,
        ],
    )
    def k(adst_h, bsrc_h, src_h, dst_h, z_h, d_v, s_v, ga, gb, sem1, sem2):
        w = _wid()

        def chunk(i, carry):
            base = w * EPW + i * K2CH
            pltpu.sync_copy(dst_h.at[pl.ds(base, K2CH)], d_v)
            pltpu.sync_copy(src_h.at[pl.ds(base, K2CH)], s_v)
            cp1 = pltpu.async_copy(adst_h.at[d_v], ga, sem1)
            cp2 = pltpu.async_copy(bsrc_h.at[s_v], gb, sem2)
            cp1.wait()
            cp2.wait()

            def addrow(r, c2):
                for cg in range(12):
                    sl = pl.ds(cg * LANES, LANES)
                    ga[r, sl] = ga[r, sl] + gb[r, sl]
                return c2

            lax.fori_loop(0, K2CH, addrow, jnp.int32(0))
            pltpu.sync_copy(ga, z_h.at[pl.ds(base, K2CH)])
            return carry

        lax.fori_loop(0, EPW // K2CH, chunk, jnp.int32(0))

    return k(adst, bsrc, src, dst)


def _k3_edge_mlp(z, ec_W2, ec_b2):
    EB = 2000

    def body(z_r, w2_r, b2_r, m_r):
        zb = z_r[...]
        w2 = w2_r[...]
        b2 = b2_r[...]
        cols = []
        for k in range(3):
            zk = jnp.maximum(zb[:, k * CC:(k + 1) * CC], 0.0)
            cols.append(jnp.dot(zk, w2[k], preferred_element_type=jnp.float32)
                        + b2[k][None, :])
        cols.append(jnp.zeros((EB, CC), jnp.float32))
        m_r[...] = jnp.concatenate(cols, axis=1)

    return pl.pallas_call(
        body,
        grid=(EE // EB,),
        in_specs=[
            pl.BlockSpec((EB, 4 * CC), lambda i: (i, 0)),
            pl.BlockSpec((3, CC, CC), lambda i: (0, 0, 0)),
            pl.BlockSpec((3, CC), lambda i: (0, 0)),
        ],
        out_specs=pl.BlockSpec((EB, 4 * CC), lambda i: (i, 0)),
        out_shape=jax.ShapeDtypeStruct((EE, 4 * CC), jnp.float32),
    )(z, ec_W2, ec_b2)


def _scan_compact(dst_h, ea_h, aux_h, sc_d, sc_e, sc_a, pk_l, lo,
                  ldst_shift, ea_shift, use_iota_aux):
    def chunk(c, off):
        base = c * SCAN_CH
        pltpu.sync_copy(dst_h.at[pl.ds(base, SCAN_CH)], sc_d)
        pltpu.sync_copy(ea_h.at[pl.ds(base, SCAN_CH)], sc_e)
        if not use_iota_aux:
            pltpu.sync_copy(aux_h.at[pl.ds(base, SCAN_CH)], sc_a)

        def grp(g, off2):
            v = sc_d[pl.ds(g * LANES, LANES)]
            eav = sc_e[pl.ds(g * LANES, LANES)]
            m = (v >= lo) & (v < lo + NT)
            mi = m.astype(jnp.int32)
            cnt = jnp.sum(mi)
            offg = jnp.minimum(off2, CAP - 16)
            pos = offg + plsc.cumsum(mi) - mi
            if use_iota_aux:
                aux = base + g * LANES + lax.iota(jnp.int32, LANES)
            else:
                aux = sc_a[pl.ds(g * LANES, LANES)]
            pk = aux + ((v - lo) << ldst_shift) + ((eav + 2) << ea_shift)
            plsc.store_scatter(pk_l, [pos], pk, mask=m)
            return off2 + cnt

        return lax.fori_loop(0, SCAN_CH // LANES, grp, off)

    off = lax.fori_loop(0, EE // SCAN_CH, chunk, jnp.int32(0))
    return jnp.minimum(off, CAP - 16)


def _k4_segmax(m_in, dst, ea, bn_gamma, bn_beta):
    NEG = jnp.float32(-jnp.inf)
    AUXM = (1 << 19) - 1

    @functools.partial(
        pl.kernel, mesh=plsc.VectorSubcoreMesh(**_MESH),
        compiler_params=_SC_PARAMS,
        out_type=[jax.ShapeDtypeStruct((NPAD, 2 * CC), jnp.float32),
                  jax.ShapeDtypeStruct((NPAD, 2 * CC), jnp.float32)],
        scratch_types=[
            pltpu.VMEM((SCAN_CH,), jnp.int32),
            pltpu.VMEM((SCAN_CH,), jnp.int32),
            pltpu.VMEM((CAP,), jnp.int32),
            pltpu.VMEM((GCH,), jnp.int32),
            pltpu.VMEM((GCH,), jnp.int32),
            pltpu.VMEM((GCH, 4 * CC), jnp.float32),
            pltpu.VMEM((GCH, 4 * CC), jnp.float32),
            pltpu.VMEM((NT, 2 * CC), jnp.float32),
            pltpu.VMEM((NT, 2 * CC), jnp.float32),
            pltpu.VMEM((3, CC), jnp.float32),
            pltpu.VMEM((3, CC), jnp.float32),
            pltpu.SemaphoreType.DMA,
            pltpu.SemaphoreType.DMA,
        ],
    )
    def k(m_h, dst_h, ea_h, g_h, b_h, x12_h, x3_h, sc_d, sc_e, pk_l,
          eidb0, eidb1, mrows0, mrows1, acc01, acc2z, gam, bet, sem0,
          sem1):
        w = _wid()
        lo = w * NT

        ninf = jnp.full((LANES,), NEG)
        zi = jnp.zeros((LANES,), jnp.int32)

        def init_r(r, c2):
            for cg in range(2 * CC // LANES):
                sl = pl.ds(cg * LANES, LANES)
                acc01[r, sl] = ninf
                acc2z[r, sl] = ninf
            return c2

        lax.fori_loop(0, NT, init_r, jnp.int32(0))

        def init_e(r, c2):
            pk_l[pl.ds(r * LANES, LANES)] = zi
            return c2

        lax.fori_loop(0, CAP // LANES, init_e, jnp.int32(0))

        nk = _scan_compact(dst_h, ea_h, None, sc_d, sc_e, None, pk_l, lo,
                           19, 28, True)
        nch = (nk + GCH - 1) // GCH
        eidbs = (eidb0, eidb1)
        mrowss = (mrows0, mrows1)
        sems = (sem0, sem1)
        ninfv = jnp.full((LANES,), NEG)

        def start_gather(j, b):
            kb = jnp.minimum(j * GCH, CAP - GCH)
            for g in range(GCH // LANES):
                pkv = pk_l[pl.ds(kb + g * LANES, LANES)]
                eidbs[b][pl.ds(g * LANES, LANES)] = pkv & AUXM
            return pltpu.async_copy(m_h.at[eidbs[b]], mrowss[b], sems[b])

        def work(j, b):
            kb = jnp.minimum(j * GCH, CAP - GCH)
            mrows = mrowss[b]

            def grp(g, c3):
                pkv = pk_l[pl.ds(kb + g * LANES, LANES)]
                for lane in range(LANES):
                    pk = pkv[lane]
                    ldst = (pk >> 19) & 511
                    ea2 = (pk >> 28) & 7
                    va = kb + g * LANES + lane < nk
                    conds = (va & (ea2 <= 2), va & (ea2 >= 2), va)
                    for kc in range(3):
                        @pl.when(conds[kc])
                        def _(kc=kc, lane=lane):
                            a = acc01 if kc < 2 else acc2z
                            ab = (kc % 2) * CC
                            for cg in range(CC // LANES):
                                sl = pl.ds(ab + cg * LANES, LANES)
                                msl = pl.ds(kc * CC + cg * LANES, LANES)
                                a[ldst, sl] = jnp.maximum(
                                    a[ldst, sl],
                                    mrows[g * LANES + lane, msl])
                return c3

            lax.fori_loop(0, GCH // LANES, grp, jnp.int32(0))

        start_gather(0, 0)
        start_gather(1, 1)
        npair = (nch + 1) // 2

        def pair(p, c2):
            j = p * 2
            pltpu.make_async_copy(m_h.at[eidb0], mrows0, sem0).wait()
            work(j, 0)
            start_gather(j + 2, 0)
            pltpu.make_async_copy(m_h.at[eidb1], mrows1, sem1).wait()
            work(j + 1, 1)
            start_gather(j + 3, 1)
            return c2

        lax.fori_loop(0, npair, pair, jnp.int32(0))
        pltpu.make_async_copy(m_h.at[eidb0], mrows0, sem0).wait()
        pltpu.make_async_copy(m_h.at[eidb1], mrows1, sem1).wait()

        pltpu.sync_copy(g_h, gam)
        pltpu.sync_copy(b_h, bet)
        zf = jnp.zeros((LANES,), jnp.float32)

        def fin_r(r, c2):
            for kc in range(3):
                a = acc01 if kc < 2 else acc2z
                cb = (kc % 2) * CC
                for cg in range(CC // LANES):
                    sl = pl.ds(cb + cg * LANES, LANES)
                    gsl = pl.ds(cg * LANES, LANES)
                    v = a[r, sl]
                    v = jnp.where(v == NEG, 0.0, v)
                    v = jnp.maximum(v * (gam[kc, gsl] * _BN_S)
                                    + bet[kc, gsl], 0.0)
                    a[r, sl] = v
            for cg in range(CC // LANES):
                acc2z[r, pl.ds(CC + cg * LANES, LANES)] = zf
            return c2

        lax.fori_loop(0, NT, fin_r, jnp.int32(0))
        pltpu.sync_copy(acc01, x12_h.at[pl.ds(lo, NT)])
        pltpu.sync_copy(acc2z, x3_h.at[pl.ds(lo, NT)])

    return k(m_in, dst, ea, bn_gamma, bn_beta)


def _k5_rgcn_sums(x12, x3, src, dst, ea):
    AUXM = (1 << 14) - 1

    @functools.partial(
        pl.kernel, mesh=plsc.VectorSubcoreMesh(**_MESH),
        compiler_params=_SC_PARAMS,
        out_type=jax.ShapeDtypeStruct((3, NPAD, 2 * CC), jnp.float32),
        scratch_types=[
            pltpu.VMEM((SCAN_CH,), jnp.int32),
            pltpu.VMEM((SCAN_CH,), jnp.int32),
            pltpu.VMEM((SCAN_CH,), jnp.int32),
            pltpu.VMEM((CAP,), jnp.int32),
            pltpu.VMEM((GCH,), jnp.int32),
            pltpu.VMEM((GCH,), jnp.int32),
            pltpu.VMEM((GCH, 2 * CC), jnp.float32),
            pltpu.VMEM((GCH, 2 * CC), jnp.float32),
            pltpu.VMEM((NT, 2 * CC), jnp.float32),
            pltpu.VMEM((NT, 2 * CC), jnp.float32),
            pltpu.SemaphoreType.DMA,
            pltpu.SemaphoreType.DMA,
        ],
    )
    def k(x12_h, x3_h, src_h, dst_h, ea_h, s_out, sc_d, sc_e, sc_s, pk_l,
          srcb0, srcb1, xrows0, xrows1, accAB, accCc, sem0, sem1):
        w = _wid()
        lo = w * NT

        zi = jnp.zeros((LANES,), jnp.int32)

        def init_e(r, c2):
            pk_l[pl.ds(r * LANES, LANES)] = zi
            return c2

        lax.fori_loop(0, CAP // LANES, init_e, jnp.int32(0))

        nk = _scan_compact(dst_h, ea_h, src_h, sc_d, sc_e, sc_s, pk_l, lo,
                           14, 23, False)
        nch = (nk + GCH - 1) // GCH

        zf = jnp.zeros((LANES,), jnp.float32)
        one0 = jnp.where(lax.iota(jnp.int32, LANES) == 0, 1.0, 0.0
                         ).astype(jnp.float32)

        def zero_acc(both):
            def init_r(r, c2):
                for cg in range(2 * CC // LANES):
                    sl = pl.ds(cg * LANES, LANES)
                    accAB[r, sl] = zf
                    if both:
                        accCc[r, sl] = zf
                return c2

            lax.fori_loop(0, NT, init_r, jnp.int32(0))

        srcbs = (srcb0, srcb1)
        xrowss = (xrows0, xrows1)
        sems = (sem0, sem1)
        il = lax.iota(jnp.int32, LANES)
        cnt_vs = tuple(jnp.where(il == q, 1.0, 0.0).astype(jnp.float32)
                       for q in range(4))

        for ps in range(2):
            zero_acc(ps == 0)
            xh = x12_h if ps == 0 else x3_h

            def start_gather(j, b, xh=xh):
                kb = jnp.minimum(j * GCH, CAP - GCH)
                for g in range(GCH // LANES):
                    pkv = pk_l[pl.ds(kb + g * LANES, LANES)]
                    srcbs[b][pl.ds(g * LANES, LANES)] = pkv & AUXM
                return pltpu.async_copy(xh.at[srcbs[b]], xrowss[b], sems[b])

            def work(j, b, ps=ps):
                kb = jnp.minimum(j * GCH, CAP - GCH)
                xrows = xrowss[b]

                def grp(g, c3):
                    pkv = pk_l[pl.ds(kb + g * LANES, LANES)]
                    for lane in range(LANES):
                        pk = pkv[lane]
                        ldst = (pk >> 14) & 511
                        ea2 = (pk >> 23) & 7
                        va = kb + g * LANES + lane < nk
                        if ps == 0:
                            combos = ((va & (ea2 == 0), accAB, 0, 0, 0),
                                      (va & ((ea2 == 1) | (ea2 == 2)),
                                       accAB, CC, 0, 1),
                                      (va & (ea2 >= 2), accCc, 0, CC, 2))
                        else:
                            combos = ((va & (ea2 == 0), accAB, 0, 0, -1),
                                      (va & (ea2 > 0), accAB, CC, 0, -1))
                        for (cond, a, ab, xb, cq) in combos:
                            @pl.when(cond)
                            def _(a=a, ab=ab, xb=xb, cq=cq, lane=lane):
                                for cg in range(CC // LANES):
                                    sl = pl.ds(ab + cg * LANES, LANES)
                                    xsl = pl.ds(xb + cg * LANES, LANES)
                                    a[ldst, sl] = (a[ldst, sl]
                                                   + xrows[g * LANES + lane,
                                                           xsl])
                                if cq >= 0:
                                    csl = pl.ds(CC, LANES)
                                    accCc[ldst, csl] = (accCc[ldst, csl]
                                                        + cnt_vs[cq])

                        if ps == 0:
                            @pl.when(va)
                            def _():
                                csl = pl.ds(CC, LANES)
                                accCc[ldst, csl] = (accCc[ldst, csl]
                                                    + cnt_vs[3])
                    return c3

                lax.fori_loop(0, GCH // LANES, grp, jnp.int32(0))

            start_gather(0, 0)
            start_gather(1, 1)
            npair = (nch + 1) // 2

            def pair(p, c2, xh=xh):
                j = p * 2
                pltpu.make_async_copy(xh.at[srcb0], xrows0, sem0).wait()
                work(j, 0)
                start_gather(j + 2, 0)
                pltpu.make_async_copy(xh.at[srcb1], xrows1, sem1).wait()
                work(j + 1, 1)
                start_gather(j + 3, 1)
                return c2

            lax.fori_loop(0, npair, pair, jnp.int32(0))
            pltpu.make_async_copy(xh.at[srcb0], xrows0, sem0).wait()
            pltpu.make_async_copy(xh.at[srcb1], xrows1, sem1).wait()
            if ps == 0:
                pltpu.sync_copy(accAB, s_out.at[0, pl.ds(lo, NT)])
                pltpu.sync_copy(accCc, s_out.at[1, pl.ds(lo, NT)])
            else:
                pltpu.sync_copy(accAB, s_out.at[2, pl.ds(lo, NT)])

    return k(x12, x3, src, dst, ea)


def _k6_combine(x12, x3, s, rg_W, rg_root, rg_bias):
    RB = 1000

    def body(x12_r, x3_r, s_r, w_r, root_r, bias_r, o_r):
        x12b = x12_r[...]
        x3b = x3_r[...]
        sb = s_r[...]
        roots = root_r[...]
        ws = w_r[...]
        bias = bias_r[...]
        xs = (x12b[:, :CC], x12b[:, CC:], x3b[:, :CC])
        out = jnp.zeros((RB, CC), jnp.float32)
        for kc in range(3):
            out = out + jnp.dot(xs[kc], roots[kc],
                                preferred_element_type=jnp.float32)
            out = out + bias[kc][None, :]
        cnt0 = jnp.maximum(sb[1, :, CC:CC + 1], 1.0)
        cnt1 = jnp.maximum(sb[1, :, CC + 1:CC + 2], 1.0)
        cnt2 = jnp.maximum(sb[1, :, CC + 2:CC + 3], 1.0)
        cnt3 = jnp.maximum(sb[1, :, CC + 3:CC + 4]
                           - sb[1, :, CC:CC + 1], 1.0)
        combos = ((sb[0, :, :CC], cnt0, 0, 0),
                  (sb[0, :, CC:], cnt1, 0, 1),
                  (sb[1, :, :CC], cnt2, 1, 1),
                  (sb[2, :, :CC], cnt0, 2, 0),
                  (sb[2, :, CC:], cnt3, 2, 1))
        for (agg, cnt, kc, r) in combos:
            out = out + jnp.dot(agg / cnt, ws[kc, r],
                                preferred_element_type=jnp.float32)
        o_r[...] = out

    return pl.pallas_call(
        body,
        grid=(NN // RB,),
        in_specs=[
            pl.BlockSpec((RB, 2 * CC), lambda i: (i, 0)),
            pl.BlockSpec((RB, 2 * CC), lambda i: (i, 0)),
            pl.BlockSpec((3, RB, 2 * CC), lambda i: (0, i, 0)),
            pl.BlockSpec((3, 2, CC, CC), lambda i: (0, 0, 0, 0)),
            pl.BlockSpec((3, CC, CC), lambda i: (0, 0, 0)),
            pl.BlockSpec((3, CC), lambda i: (0, 0)),
        ],
        out_specs=pl.BlockSpec((RB, CC), lambda i: (i, 0)),
        out_shape=jax.ShapeDtypeStruct((NN, CC), jnp.float32),
    )(x12, x3, s, rg_W, rg_root, rg_bias)


def kernel(x, edge_index, edge_attr, W011, b011, gamma0, beta0, ec_W1,
           ec_b1, ec_W2, ec_b2, bn_gamma, bn_beta, rg_W, rg_root, rg_bias):
    src = edge_index[0].astype(jnp.int32)
    dst = edge_index[1].astype(jnp.int32)
    ea = edge_attr.astype(jnp.int32)

    adst, bsrc = _k1_node_tables(x, W011, b011, gamma0, beta0, ec_W1, ec_b1)
    z = _k2_edge_z(adst, bsrc, src, dst)
    m = _k3_edge_mlp(z, ec_W2, ec_b2)
    x12, x3 = _k4_segmax(m, dst, ea, bn_gamma, bn_beta)
    s = _k5_rgcn_sums(x12, x3, src, dst, ea)
    return _k6_combine(x12, x3, s, rg_W, rg_root, rg_bias)


# speedup vs baseline: 1.1212x; 1.0479x over previous
import functools
import jax
import jax.numpy as jnp
from jax import lax
from jax.experimental import pallas as pl
from jax.experimental.pallas import tpu as pltpu
from jax.experimental.pallas import tpu_sc as plsc

NN = 10000
EE = 320000
DIN = 128
CC = 64
NC, NS, LANES = 2, 16, 16
NW = NC * NS
NT = 320
NPAD = NW * NT
CAP = 12288
SCAN_CH = 1600
GCH = 48
EPW = EE // NW
K2CH = 80

_BN_S = float(1.0 / (1.0 + 1e-5) ** 0.5)

_MESH = dict(core_axis_name="c", subcore_axis_name="s",
             num_cores=NC, num_subcores=NS)
_SC_PARAMS = pltpu.CompilerParams(needs_layout_passes=False)


def _wid():
    return lax.axis_index("s") * NC + lax.axis_index("c")


def _k1_node_tables(x, W011, b011, gamma0, beta0, ec_W1, ec_b1):
    RB = 1000

    def body(x_r, w_r, b_r, g_r, be_r, w1_r, b1_r, a_r, bb_r):
        h = jnp.dot(x_r[...], w_r[...], preferred_element_type=jnp.float32)
        h = h + b_r[...]
        h = h * (g_r[...] * _BN_S) + be_r[...]
        h = jnp.maximum(h, 0.0)
        w1 = w1_r[...]
        b1 = b1_r[...]
        acols = []
        bcols = []
        for k in range(3):
            w1a = w1[k, :CC, :]
            w1b = w1[k, CC:, :]
            acols.append(jnp.dot(h, w1a - w1b,
                                 preferred_element_type=jnp.float32)
                         + b1[k][None, :])
            bcols.append(jnp.dot(h, w1b, preferred_element_type=jnp.float32))
        z = jnp.zeros((RB, CC), jnp.float32)
        a_r[...] = jnp.concatenate(acols + [z], axis=1)
        bb_r[...] = jnp.concatenate(bcols + [z], axis=1)

    return pl.pallas_call(
        body,
        grid=(NN // RB,),
        in_specs=[
            pl.BlockSpec((RB, DIN), lambda i: (i, 0)),
            pl.BlockSpec((DIN, CC), lambda i: (0, 0)),
            pl.BlockSpec((1, CC), lambda i: (0, 0)),
            pl.BlockSpec((1, CC), lambda i: (0, 0)),
            pl.BlockSpec((1, CC), lambda i: (0, 0)),
            pl.BlockSpec((3, 2 * CC, CC), lambda i: (0, 0, 0)),
            pl.BlockSpec((3, CC), lambda i: (0, 0)),
        ],
        out_specs=[
            pl.BlockSpec((RB, 4 * CC), lambda i: (i, 0)),
            pl.BlockSpec((RB, 4 * CC), lambda i: (i, 0)),
        ],
        out_shape=[
            jax.ShapeDtypeStruct((NN, 4 * CC), jnp.float32),
            jax.ShapeDtypeStruct((NN, 4 * CC), jnp.float32),
        ],
    )(x, W011, b011.reshape(1, CC), gamma0.reshape(1, CC),
      beta0.reshape(1, CC), ec_W1, ec_b1)


def _k2_edge_z(adst, bsrc, src, dst):
    @functools.partial(
        pl.kernel, mesh=plsc.VectorSubcoreMesh(**_MESH),
        compiler_params=_SC_PARAMS,
        out_type=jax.ShapeDtypeStruct((EE, 4 * CC), jnp.float32),
        scratch_types=[
            pltpu.VMEM((K2CH,), jnp.int32),
            pltpu.VMEM((K2CH,), jnp.int32),
            pltpu.VMEM((K2CH,), jnp.int32),
            pltpu.VMEM((K2CH,), jnp.int32),
            pltpu.VMEM((K2CH, 4 * CC), jnp.float32),
            pltpu.VMEM((K2CH, 4 * CC), jnp.float32),
            pltpu.VMEM((K2CH, 4 * CC), jnp.float32),
            pltpu.VMEM((K2CH, 4 * CC), jnp.float32),
            pltpu.SemaphoreType.DMA,
            pltpu.SemaphoreType.DMA,
            pltpu.SemaphoreType.DMA,
            pltpu.SemaphoreType.DMA,
            pltpu.SemaphoreType.DMA,
            pltpu.SemaphoreType.DMA,
        ],
    )
    def k(adst_h, bsrc_h, src_h, dst_h, z_h, d0, s0, d1, s1, ga0, gb0,
          ga1, gb1, semA0, semB0, semA1, semB1, semW0, semW1):
        w = _wid()
        nch2 = EPW // K2CH
        ds_ = (d0, d1)
        ss_ = (s0, s1)
        gas = (ga0, ga1)
        gbs = (gb0, gb1)
        semA = (semA0, semA1)
        semB = (semB0, semB1)
        semW = (semW0, semW1)

        def zslice(i):
            ic = jnp.minimum(i, nch2 - 1)
            return z_h.at[pl.ds(w * EPW + ic * K2CH, K2CH)]

        def start(i, b, wait_write):
            ic = jnp.minimum(i, nch2 - 1)
            base = w * EPW + ic * K2CH
            if wait_write:
                pltpu.make_async_copy(gas[b], zslice(i - 2), semW[b]).wait()
            pltpu.sync_copy(dst_h.at[pl.ds(base, K2CH)], ds_[b])
            pltpu.sync_copy(src_h.at[pl.ds(base, K2CH)], ss_[b])
            pltpu.async_copy(adst_h.at[ds_[b]], gas[b], semA[b])
            pltpu.async_copy(bsrc_h.at[ss_[b]], gbs[b], semB[b])

        def work(i, b):
            pltpu.make_async_copy(adst_h.at[ds_[b]], gas[b],
                                  semA[b]).wait()
            pltpu.make_async_copy(bsrc_h.at[ss_[b]], gbs[b],
                                  semB[b]).wait()
            ga = gas[b]
            gb = gbs[b]

            def addrow(r, c2):
                for cg in range(12):
                    sl = pl.ds(cg * LANES, LANES)
                    ga[r, sl] = ga[r, sl] + gb[r, sl]
                return c2

            lax.fori_loop(0, K2CH, addrow, jnp.int32(0))
            pltpu.async_copy(ga, zslice(i), semW[b])

        start(0, 0, False)
        start(1, 1, False)
        work(0, 0)
        start(2, 0, True)
        work(1, 1)
        start(3, 1, True)

        def pair(p, c2):
            i = (p + 1) * 2
            work(i, 0)
            start(i + 2, 0, True)
            work(i + 1, 1)
            start(i + 3, 1, True)
            return c2

        lax.fori_loop(0, nch2 // 2 - 1, pair, jnp.int32(0))
        work(nch2 - 1, 0)
        pltpu.make_async_copy(adst_h.at[d1], ga1, semA1).wait()
        pltpu.make_async_copy(bsrc_h.at[s1], gb1, semB1).wait()
        pltpu.make_async_copy(ga0, zslice(nch2 - 1), semW0).wait()

    return k(adst, bsrc, src, dst)


def _k3_edge_mlp(z, ec_W2, ec_b2):
    EB = 2000

    def body(z_r, w2_r, b2_r, m_r):
        zb = z_r[...]
        w2 = w2_r[...]
        b2 = b2_r[...]
        cols = []
        for k in range(3):
            zk = jnp.maximum(zb[:, k * CC:(k + 1) * CC], 0.0)
            cols.append(jnp.dot(zk, w2[k], preferred_element_type=jnp.float32)
                        + b2[k][None, :])
        cols.append(jnp.zeros((EB, CC), jnp.float32))
        m_r[...] = jnp.concatenate(cols, axis=1)

    return pl.pallas_call(
        body,
        grid=(EE // EB,),
        in_specs=[
            pl.BlockSpec((EB, 4 * CC), lambda i: (i, 0)),
            pl.BlockSpec((3, CC, CC), lambda i: (0, 0, 0)),
            pl.BlockSpec((3, CC), lambda i: (0, 0)),
        ],
        out_specs=pl.BlockSpec((EB, 4 * CC), lambda i: (i, 0)),
        out_shape=jax.ShapeDtypeStruct((EE, 4 * CC), jnp.float32),
    )(z, ec_W2, ec_b2)


def _scan_compact(dst_h, ea_h, aux_h, sc_d, sc_e, sc_a, pk_l, lo,
                  ldst_shift, ea_shift, use_iota_aux):
    def chunk(c, off):
        base = c * SCAN_CH
        pltpu.sync_copy(dst_h.at[pl.ds(base, SCAN_CH)], sc_d)
        pltpu.sync_copy(ea_h.at[pl.ds(base, SCAN_CH)], sc_e)
        if not use_iota_aux:
            pltpu.sync_copy(aux_h.at[pl.ds(base, SCAN_CH)], sc_a)

        def grp2(g2, off2):
            datas = []
            for u in range(2):
                g = g2 * 2 + u
                v = sc_d[pl.ds(g * LANES, LANES)]
                eav = sc_e[pl.ds(g * LANES, LANES)]
                m = (v >= lo) & (v < lo + NT)
                mi = m.astype(jnp.int32)
                cs = plsc.cumsum(mi)
                if use_iota_aux:
                    aux = base + g * LANES + lax.iota(jnp.int32, LANES)
                else:
                    aux = sc_a[pl.ds(g * LANES, LANES)]
                pk = (aux + ((v - lo) << ldst_shift)
                      + ((eav + 2) << ea_shift))
                datas.append((m, mi, cs, pk))
            m0, mi0, cs0, pk0 = datas[0]
            m1, mi1, cs1, pk1 = datas[1]
            cnt0 = cs0[LANES - 1]
            cnt1 = cs1[LANES - 1]
            offg = jnp.minimum(off2, CAP - 16)
            plsc.store_scatter(pk_l, [offg + cs0 - mi0], pk0, mask=m0)
            offh = jnp.minimum(off2 + cnt0, CAP - 16)
            plsc.store_scatter(pk_l, [offh + cs1 - mi1], pk1, mask=m1)
            return off2 + cnt0 + cnt1

        return lax.fori_loop(0, SCAN_CH // LANES // 2, grp2, off)

    off = lax.fori_loop(0, EE // SCAN_CH, chunk, jnp.int32(0))
    return jnp.minimum(off, CAP - 16)


def _k4_segmax(m_in, dst, ea, bn_gamma, bn_beta):
    NEG = jnp.float32(-jnp.inf)
    AUXM = (1 << 19) - 1

    @functools.partial(
        pl.kernel, mesh=plsc.VectorSubcoreMesh(**_MESH),
        compiler_params=_SC_PARAMS,
        out_type=[jax.ShapeDtypeStruct((NPAD, 2 * CC), jnp.float32),
                  jax.ShapeDtypeStruct((NPAD, 2 * CC), jnp.float32)],
        scratch_types=[
            pltpu.VMEM((SCAN_CH,), jnp.int32),
            pltpu.VMEM((SCAN_CH,), jnp.int32),
            pltpu.VMEM((CAP,), jnp.int32),
            pltpu.VMEM((GCH,), jnp.int32),
            pltpu.VMEM((GCH,), jnp.int32),
            pltpu.VMEM((GCH, 4 * CC), jnp.float32),
            pltpu.VMEM((GCH, 4 * CC), jnp.float32),
            pltpu.VMEM((NT, 2 * CC), jnp.float32),
            pltpu.VMEM((NT, 2 * CC), jnp.float32),
            pltpu.VMEM((3, CC), jnp.float32),
            pltpu.VMEM((3, CC), jnp.float32),
            pltpu.SemaphoreType.DMA,
            pltpu.SemaphoreType.DMA,
        ],
    )
    def k(m_h, dst_h, ea_h, g_h, b_h, x12_h, x3_h, sc_d, sc_e, pk_l,
          eidb0, eidb1, mrows0, mrows1, acc01, acc2z, gam, bet, sem0,
          sem1):
        w = _wid()
        lo = w * NT

        ninf = jnp.full((LANES,), NEG)
        zi = jnp.zeros((LANES,), jnp.int32)

        def init_r(r, c2):
            for cg in range(2 * CC // LANES):
                sl = pl.ds(cg * LANES, LANES)
                acc01[r, sl] = ninf
                acc2z[r, sl] = ninf
            return c2

        lax.fori_loop(0, NT, init_r, jnp.int32(0))

        def init_e(r, c2):
            pk_l[pl.ds(r * LANES, LANES)] = zi
            return c2

        lax.fori_loop(0, CAP // LANES, init_e, jnp.int32(0))

        nk = _scan_compact(dst_h, ea_h, None, sc_d, sc_e, None, pk_l, lo,
                           19, 28, True)
        nch = (nk + GCH - 1) // GCH
        eidbs = (eidb0, eidb1)
        mrowss = (mrows0, mrows1)
        sems = (sem0, sem1)
        ninfv = jnp.full((LANES,), NEG)

        def start_gather(j, b):
            kb = jnp.minimum(j * GCH, CAP - GCH)
            for g in range(GCH // LANES):
                pkv = pk_l[pl.ds(kb + g * LANES, LANES)]
                eidbs[b][pl.ds(g * LANES, LANES)] = pkv & AUXM
            return pltpu.async_copy(m_h.at[eidbs[b]], mrowss[b], sems[b])

        def work(j, b):
            kb = jnp.minimum(j * GCH, CAP - GCH)
            mrows = mrowss[b]

            def grp(g, c3):
                pkv = pk_l[pl.ds(kb + g * LANES, LANES)]
                for lane in range(LANES):
                    pk = pkv[lane]
                    ldst = (pk >> 19) & 511
                    ea2 = (pk >> 28) & 7
                    va = kb + g * LANES + lane < nk
                    conds = (va & (ea2 <= 2), va & (ea2 >= 2), va)
                    for kc in range(3):
                        @pl.when(conds[kc])
                        def _(kc=kc, lane=lane):
                            a = acc01 if kc < 2 else acc2z
                            ab = (kc % 2) * CC
                            for cg in range(CC // LANES):
                                sl = pl.ds(ab + cg * LANES, LANES)
                                msl = pl.ds(kc * CC + cg * LANES, LANES)
                                a[ldst, sl] = jnp.maximum(
                                    a[ldst, sl],
                                    mrows[g * LANES + lane, msl])
                return c3

            lax.fori_loop(0, GCH // LANES, grp, jnp.int32(0))

        start_gather(0, 0)
        start_gather(1, 1)
        npair = (nch + 1) // 2

        def pair(p, c2):
            j = p * 2
            pltpu.make_async_copy(m_h.at[eidb0], mrows0, sem0).wait()
            work(j, 0)
            start_gather(j + 2, 0)
            pltpu.make_async_copy(m_h.at[eidb1], mrows1, sem1).wait()
            work(j + 1, 1)
            start_gather(j + 3, 1)
            return c2

        lax.fori_loop(0, npair, pair, jnp.int32(0))
        pltpu.make_async_copy(m_h.at[eidb0], mrows0, sem0).wait()
        pltpu.make_async_copy(m_h.at[eidb1], mrows1, sem1).wait()

        pltpu.sync_copy(g_h, gam)
        pltpu.sync_copy(b_h, bet)
        zf = jnp.zeros((LANES,), jnp.float32)

        def fin_r(r, c2):
            for kc in range(3):
                a = acc01 if kc < 2 else acc2z
                cb = (kc % 2) * CC
                for cg in range(CC // LANES):
                    sl = pl.ds(cb + cg * LANES, LANES)
                    gsl = pl.ds(cg * LANES, LANES)
                    v = a[r, sl]
                    v = jnp.where(v == NEG, 0.0, v)
                    v = jnp.maximum(v * (gam[kc, gsl] * _BN_S)
                                    + bet[kc, gsl], 0.0)
                    a[r, sl] = v
            for cg in range(CC // LANES):
                acc2z[r, pl.ds(CC + cg * LANES, LANES)] = zf
            return c2

        lax.fori_loop(0, NT, fin_r, jnp.int32(0))
        pltpu.sync_copy(acc01, x12_h.at[pl.ds(lo, NT)])
        pltpu.sync_copy(acc2z, x3_h.at[pl.ds(lo, NT)])

    return k(m_in, dst, ea, bn_gamma, bn_beta)


def _k5_rgcn_sums(x12, x3, src, dst, ea):
    AUXM = (1 << 14) - 1

    @functools.partial(
        pl.kernel, mesh=plsc.VectorSubcoreMesh(**_MESH),
        compiler_params=_SC_PARAMS,
        out_type=jax.ShapeDtypeStruct((3, NPAD, 2 * CC), jnp.float32),
        scratch_types=[
            pltpu.VMEM((SCAN_CH,), jnp.int32),
            pltpu.VMEM((SCAN_CH,), jnp.int32),
            pltpu.VMEM((SCAN_CH,), jnp.int32),
            pltpu.VMEM((CAP,), jnp.int32),
            pltpu.VMEM((GCH,), jnp.int32),
            pltpu.VMEM((GCH,), jnp.int32),
            pltpu.VMEM((GCH, 2 * CC), jnp.float32),
            pltpu.VMEM((GCH, 2 * CC), jnp.float32),
            pltpu.VMEM((NT, 2 * CC), jnp.float32),
            pltpu.VMEM((NT, 2 * CC), jnp.float32),
            pltpu.SemaphoreType.DMA,
            pltpu.SemaphoreType.DMA,
        ],
    )
    def k(x12_h, x3_h, src_h, dst_h, ea_h, s_out, sc_d, sc_e, sc_s, pk_l,
          srcb0, srcb1, xrows0, xrows1, accAB, accCc, sem0, sem1):
        w = _wid()
        lo = w * NT

        zi = jnp.zeros((LANES,), jnp.int32)

        def init_e(r, c2):
            pk_l[pl.ds(r * LANES, LANES)] = zi
            return c2

        lax.fori_loop(0, CAP // LANES, init_e, jnp.int32(0))

        nk = _scan_compact(dst_h, ea_h, src_h, sc_d, sc_e, sc_s, pk_l, lo,
                           14, 23, False)
        nch = (nk + GCH - 1) // GCH

        zf = jnp.zeros((LANES,), jnp.float32)
        one0 = jnp.where(lax.iota(jnp.int32, LANES) == 0, 1.0, 0.0
                         ).astype(jnp.float32)

        def zero_acc(both):
            def init_r(r, c2):
                for cg in range(2 * CC // LANES):
                    sl = pl.ds(cg * LANES, LANES)
                    accAB[r, sl] = zf
                    if both:
                        accCc[r, sl] = zf
                return c2

            lax.fori_loop(0, NT, init_r, jnp.int32(0))

        srcbs = (srcb0, srcb1)
        xrowss = (xrows0, xrows1)
        sems = (sem0, sem1)
        il = lax.iota(jnp.int32, LANES)
        cnt_vs = tuple(jnp.where(il == q, 1.0, 0.0).astype(jnp.float32)
                       for q in range(4))

        for ps in range(2):
            zero_acc(ps == 0)
            xh = x12_h if ps == 0 else x3_h

            def start_gather(j, b, xh=xh):
                kb = jnp.minimum(j * GCH, CAP - GCH)
                for g in range(GCH // LANES):
                    pkv = pk_l[pl.ds(kb + g * LANES, LANES)]
                    srcbs[b][pl.ds(g * LANES, LANES)] = pkv & AUXM
                return pltpu.async_copy(xh.at[srcbs[b]], xrowss[b], sems[b])

            def work(j, b, ps=ps):
                kb = jnp.minimum(j * GCH, CAP - GCH)
                xrows = xrowss[b]

                def grp(g, c3):
                    pkv = pk_l[pl.ds(kb + g * LANES, LANES)]
                    for lane in range(LANES):
                        pk = pkv[lane]
                        ldst = (pk >> 14) & 511
                        ea2 = (pk >> 23) & 7
                        va = kb + g * LANES + lane < nk
                        if ps == 0:
                            combos = ((va & (ea2 == 0), accAB, 0, 0, 0),
                                      (va & ((ea2 == 1) | (ea2 == 2)),
                                       accAB, CC, 0, 1),
                                      (va & (ea2 >= 2), accCc, 0, CC, 2))
                        else:
                            combos = ((va & (ea2 == 0), accAB, 0, 0, -1),
                                      (va & (ea2 > 0), accAB, CC, 0, -1))
                        for (cond, a, ab, xb, cq) in combos:
                            @pl.when(cond)
                            def _(a=a, ab=ab, xb=xb, cq=cq, lane=lane):
                                for cg in range(CC // LANES):
                                    sl = pl.ds(ab + cg * LANES, LANES)
                                    xsl = pl.ds(xb + cg * LANES, LANES)
                                    a[ldst, sl] = (a[ldst, sl]
                                                   + xrows[g * LANES + lane,
                                                           xsl])
                                if cq >= 0:
                                    csl = pl.ds(CC, LANES)
                                    accCc[ldst, csl] = (accCc[ldst, csl]
                                                        + cnt_vs[cq])

                        if ps == 0:
                            @pl.when(va)
                            def _():
                                csl = pl.ds(CC, LANES)
                                accCc[ldst, csl] = (accCc[ldst, csl]
                                                    + cnt_vs[3])
                    return c3

                lax.fori_loop(0, GCH // LANES, grp, jnp.int32(0))

            start_gather(0, 0)
            start_gather(1, 1)
            npair = (nch + 1) // 2

            def pair(p, c2, xh=xh):
                j = p * 2
                pltpu.make_async_copy(xh.at[srcb0], xrows0, sem0).wait()
                work(j, 0)
                start_gather(j + 2, 0)
                pltpu.make_async_copy(xh.at[srcb1], xrows1, sem1).wait()
                work(j + 1, 1)
                start_gather(j + 3, 1)
                return c2

            lax.fori_loop(0, npair, pair, jnp.int32(0))
            pltpu.make_async_copy(xh.at[srcb0], xrows0, sem0).wait()
            pltpu.make_async_copy(xh.at[srcb1], xrows1, sem1).wait()
            if ps == 0:
                pltpu.sync_copy(accAB, s_out.at[0, pl.ds(lo, NT)])
                pltpu.sync_copy(accCc, s_out.at[1, pl.ds(lo, NT)])
            else:
                pltpu.sync_copy(accAB, s_out.at[2, pl.ds(lo, NT)])

    return k(x12, x3, src, dst, ea)


def _k6_combine(x12, x3, s, rg_W, rg_root, rg_bias):
    RB = 1000

    def body(x12_r, x3_r, s_r, w_r, root_r, bias_r, o_r):
        x12b = x12_r[...]
        x3b = x3_r[...]
        sb = s_r[...]
        roots = root_r[...]
        ws = w_r[...]
        bias = bias_r[...]
        xs = (x12b[:, :CC], x12b[:, CC:], x3b[:, :CC])
        out = jnp.zeros((RB, CC), jnp.float32)
        for kc in range(3):
            out = out + jnp.dot(xs[kc], roots[kc],
                                preferred_element_type=jnp.float32)
            out = out + bias[kc][None, :]
        cnt0 = jnp.maximum(sb[1, :, CC:CC + 1], 1.0)
        cnt1 = jnp.maximum(sb[1, :, CC + 1:CC + 2], 1.0)
        cnt2 = jnp.maximum(sb[1, :, CC + 2:CC + 3], 1.0)
        cnt3 = jnp.maximum(sb[1, :, CC + 3:CC + 4]
                           - sb[1, :, CC:CC + 1], 1.0)
        combos = ((sb[0, :, :CC], cnt0, 0, 0),
                  (sb[0, :, CC:], cnt1, 0, 1),
                  (sb[1, :, :CC], cnt2, 1, 1),
                  (sb[2, :, :CC], cnt0, 2, 0),
                  (sb[2, :, CC:], cnt3, 2, 1))
        for (agg, cnt, kc, r) in combos:
            out = out + jnp.dot(agg / cnt, ws[kc, r],
                                preferred_element_type=jnp.float32)
        o_r[...] = out

    return pl.pallas_call(
        body,
        grid=(NN // RB,),
        in_specs=[
            pl.BlockSpec((RB, 2 * CC), lambda i: (i, 0)),
            pl.BlockSpec((RB, 2 * CC), lambda i: (i, 0)),
            pl.BlockSpec((3, RB, 2 * CC), lambda i: (0, i, 0)),
            pl.BlockSpec((3, 2, CC, CC), lambda i: (0, 0, 0, 0)),
            pl.BlockSpec((3, CC, CC), lambda i: (0, 0, 0)),
            pl.BlockSpec((3, CC), lambda i: (0, 0)),
        ],
        out_specs=pl.BlockSpec((RB, CC), lambda i: (i, 0)),
        out_shape=jax.ShapeDtypeStruct((NN, CC), jnp.float32),
    )(x12, x3, s, rg_W, rg_root, rg_bias)


def kernel(x, edge_index, edge_attr, W011, b011, gamma0, beta0, ec_W1,
           ec_b1, ec_W2, ec_b2, bn_gamma, bn_beta, rg_W, rg_root, rg_bias):
    src = edge_index[0].astype(jnp.int32)
    dst = edge_index[1].astype(jnp.int32)
    ea = edge_attr.astype(jnp.int32)

    adst, bsrc = _k1_node_tables(x, W011, b011, gamma0, beta0, ec_W1, ec_b1)
    z = _k2_edge_z(adst, bsrc, src, dst)
    m = _k3_edge_mlp(z, ec_W2, ec_b2)
    x12, x3 = _k4_segmax(m, dst, ea, bn_gamma, bn_beta)
    s = _k5_rgcn_sums(x12, x3, src, dst, ea)
    return _k6_combine(x12, x3, s, rg_W, rg_root, rg_bias)


# speedup vs baseline: 1.3072x; 1.1659x over previous
import functools
import jax
import jax.numpy as jnp
from jax import lax
from jax.experimental import pallas as pl
from jax.experimental.pallas import tpu as pltpu
from jax.experimental.pallas import tpu_sc as plsc

NN = 10000
EE = 320000
DIN = 128
CC = 64
NC, NS, LANES = 2, 16, 16
NW = NC * NS
NT = 320
NPAD = NW * NT
CAP = 12288
SCAN_CH = 1600
GCH = 48
EPW = EE // NW
K2CH = 80

_BN_S = float(1.0 / (1.0 + 1e-5) ** 0.5)

_MESH = dict(core_axis_name="c", subcore_axis_name="s",
             num_cores=NC, num_subcores=NS)
_SC_PARAMS = pltpu.CompilerParams(needs_layout_passes=False)


def _wid():
    return lax.axis_index("s") * NC + lax.axis_index("c")


def _k1_node_tables(x, W011, b011, gamma0, beta0, ec_W1, ec_b1):
    RB = 1000

    def body(x_r, w_r, b_r, g_r, be_r, w1_r, b1_r, a_r, bb_r):
        h = jnp.dot(x_r[...], w_r[...], preferred_element_type=jnp.float32)
        h = h + b_r[...]
        h = h * (g_r[...] * _BN_S) + be_r[...]
        h = jnp.maximum(h, 0.0)
        w1 = w1_r[...]
        b1 = b1_r[...]
        acols = []
        bcols = []
        for k in range(3):
            w1a = w1[k, :CC, :]
            w1b = w1[k, CC:, :]
            acols.append(jnp.dot(h, w1a - w1b,
                                 preferred_element_type=jnp.float32)
                         + b1[k][None, :])
            bcols.append(jnp.dot(h, w1b, preferred_element_type=jnp.float32))
        z = jnp.zeros((RB, CC), jnp.float32)
        a_r[...] = jnp.concatenate(acols + [z], axis=1)
        bb_r[...] = jnp.concatenate(bcols + [z], axis=1)

    return pl.pallas_call(
        body,
        grid=(NN // RB,),
        in_specs=[
            pl.BlockSpec((RB, DIN), lambda i: (i, 0)),
            pl.BlockSpec((DIN, CC), lambda i: (0, 0)),
            pl.BlockSpec((1, CC), lambda i: (0, 0)),
            pl.BlockSpec((1, CC), lambda i: (0, 0)),
            pl.BlockSpec((1, CC), lambda i: (0, 0)),
            pl.BlockSpec((3, 2 * CC, CC), lambda i: (0, 0, 0)),
            pl.BlockSpec((3, CC), lambda i: (0, 0)),
        ],
        out_specs=[
            pl.BlockSpec((RB, 4 * CC), lambda i: (i, 0)),
            pl.BlockSpec((RB, 4 * CC), lambda i: (i, 0)),
        ],
        out_shape=[
            jax.ShapeDtypeStruct((NN, 4 * CC), jnp.float32),
            jax.ShapeDtypeStruct((NN, 4 * CC), jnp.float32),
        ],
    )(x, W011, b011.reshape(1, CC), gamma0.reshape(1, CC),
      beta0.reshape(1, CC), ec_W1, ec_b1)


def _k2_edge_z(adst, bsrc, src, dst):
    @functools.partial(
        pl.kernel, mesh=plsc.VectorSubcoreMesh(**_MESH),
        compiler_params=_SC_PARAMS,
        out_type=jax.ShapeDtypeStruct((EE, 4 * CC), jnp.float32),
        scratch_types=[
            pltpu.VMEM((K2CH,), jnp.int32),
            pltpu.VMEM((K2CH,), jnp.int32),
            pltpu.VMEM((K2CH,), jnp.int32),
            pltpu.VMEM((K2CH,), jnp.int32),
            pltpu.VMEM((K2CH, 4 * CC), jnp.float32),
            pltpu.VMEM((K2CH, 4 * CC), jnp.float32),
            pltpu.VMEM((K2CH, 4 * CC), jnp.float32),
            pltpu.VMEM((K2CH, 4 * CC), jnp.float32),
            pltpu.SemaphoreType.DMA,
            pltpu.SemaphoreType.DMA,
            pltpu.SemaphoreType.DMA,
            pltpu.SemaphoreType.DMA,
            pltpu.SemaphoreType.DMA,
            pltpu.SemaphoreType.DMA,
        ],
    )
    def k(adst_h, bsrc_h, src_h, dst_h, z_h, d0, s0, d1, s1, ga0, gb0,
          ga1, gb1, semA0, semB0, semA1, semB1, semW0, semW1):
        w = _wid()
        nch2 = EPW // K2CH
        ds_ = (d0, d1)
        ss_ = (s0, s1)
        gas = (ga0, ga1)
        gbs = (gb0, gb1)
        semA = (semA0, semA1)
        semB = (semB0, semB1)
        semW = (semW0, semW1)

        def zslice(i):
            ic = jnp.minimum(i, nch2 - 1)
            return z_h.at[pl.ds(w * EPW + ic * K2CH, K2CH)]

        def start(i, b, wait_write):
            ic = jnp.minimum(i, nch2 - 1)
            base = w * EPW + ic * K2CH
            if wait_write:
                pltpu.make_async_copy(gas[b], zslice(i - 2), semW[b]).wait()
            pltpu.sync_copy(dst_h.at[pl.ds(base, K2CH)], ds_[b])
            pltpu.sync_copy(src_h.at[pl.ds(base, K2CH)], ss_[b])
            pltpu.async_copy(adst_h.at[ds_[b]], gas[b], semA[b])
            pltpu.async_copy(bsrc_h.at[ss_[b]], gbs[b], semB[b])

        def work(i, b):
            pltpu.make_async_copy(adst_h.at[ds_[b]], gas[b],
                                  semA[b]).wait()
            pltpu.make_async_copy(bsrc_h.at[ss_[b]], gbs[b],
                                  semB[b]).wait()
            ga = gas[b]
            gb = gbs[b]

            def addrow(r, c2):
                for cg in range(12):
                    sl = pl.ds(cg * LANES, LANES)
                    ga[r, sl] = ga[r, sl] + gb[r, sl]
                return c2

            lax.fori_loop(0, K2CH, addrow, jnp.int32(0))
            pltpu.async_copy(ga, zslice(i), semW[b])

        start(0, 0, False)
        start(1, 1, False)
        work(0, 0)
        start(2, 0, True)
        work(1, 1)
        start(3, 1, True)

        def pair(p, c2):
            i = (p + 1) * 2
            work(i, 0)
            start(i + 2, 0, True)
            work(i + 1, 1)
            start(i + 3, 1, True)
            return c2

        lax.fori_loop(0, nch2 // 2 - 1, pair, jnp.int32(0))
        work(nch2 - 1, 0)
        pltpu.make_async_copy(adst_h.at[d1], ga1, semA1).wait()
        pltpu.make_async_copy(bsrc_h.at[s1], gb1, semB1).wait()
        pltpu.make_async_copy(ga0, zslice(nch2 - 1), semW0).wait()

    return k(adst, bsrc, src, dst)


def _k3_edge_mlp(z, ec_W2, ec_b2):
    EB = 2000

    def body(z_r, w2_r, b2_r, m_r):
        zb = z_r[...]
        w2 = w2_r[...]
        b2 = b2_r[...]
        cols = []
        for k in range(3):
            zk = jnp.maximum(zb[:, k * CC:(k + 1) * CC], 0.0)
            cols.append(jnp.dot(zk, w2[k], preferred_element_type=jnp.float32)
                        + b2[k][None, :])
        cols.append(jnp.zeros((EB, CC), jnp.float32))
        m_r[...] = jnp.concatenate(cols, axis=1)

    return pl.pallas_call(
        body,
        grid=(EE // EB,),
        in_specs=[
            pl.BlockSpec((EB, 4 * CC), lambda i: (i, 0)),
            pl.BlockSpec((3, CC, CC), lambda i: (0, 0, 0)),
            pl.BlockSpec((3, CC), lambda i: (0, 0)),
        ],
        out_specs=pl.BlockSpec((EB, 4 * CC), lambda i: (i, 0)),
        out_shape=jax.ShapeDtypeStruct((EE, 4 * CC), jnp.float32),
    )(z, ec_W2, ec_b2)


def _scan_compact(dst_h, ea_h, aux_h, sc_ds, sc_es, sc_as, scse, pk_l,
                  lo, ldst_shift, ea_shift, use_iota_aux):
    NCHS = EE // SCAN_CH

    def start(c, b):
        cc_ = jnp.minimum(c, NCHS - 1)
        base = cc_ * SCAN_CH
        pltpu.async_copy(dst_h.at[pl.ds(base, SCAN_CH)], sc_ds[b],
                         scse[b][0])
        pltpu.async_copy(ea_h.at[pl.ds(base, SCAN_CH)], sc_es[b],
                         scse[b][1])
        if not use_iota_aux:
            pltpu.async_copy(aux_h.at[pl.ds(base, SCAN_CH)], sc_as[b],
                             scse[b][2])

    def wait(c, b):
        cc_ = jnp.minimum(c, NCHS - 1)
        base = cc_ * SCAN_CH
        pltpu.make_async_copy(dst_h.at[pl.ds(base, SCAN_CH)], sc_ds[b],
                              scse[b][0]).wait()
        pltpu.make_async_copy(ea_h.at[pl.ds(base, SCAN_CH)], sc_es[b],
                              scse[b][1]).wait()
        if not use_iota_aux:
            pltpu.make_async_copy(aux_h.at[pl.ds(base, SCAN_CH)], sc_as[b],
                                  scse[b][2]).wait()

    def process(c, b, off):
        base = c * SCAN_CH
        sc_d = sc_ds[b]
        sc_e = sc_es[b]
        sc_a = sc_as[b] if not use_iota_aux else None

        def grp2(g2, off2):
            datas = []
            for u in range(2):
                g = g2 * 2 + u
                v = sc_d[pl.ds(g * LANES, LANES)]
                eav = sc_e[pl.ds(g * LANES, LANES)]
                m = (v >= lo) & (v < lo + NT)
                mi = m.astype(jnp.int32)
                cs = plsc.cumsum(mi)
                if use_iota_aux:
                    aux = base + g * LANES + lax.iota(jnp.int32, LANES)
                else:
                    aux = sc_a[pl.ds(g * LANES, LANES)]
                pk = (aux + ((v - lo) << ldst_shift)
                      + ((eav + 2) << ea_shift))
                datas.append((m, mi, cs, pk))
            m0, mi0, cs0, pk0 = datas[0]
            m1, mi1, cs1, pk1 = datas[1]
            cnt0 = cs0[LANES - 1]
            cnt1 = cs1[LANES - 1]
            offg = jnp.minimum(off2, CAP - 16)
            plsc.store_scatter(pk_l, [offg + cs0 - mi0], pk0, mask=m0)
            offh = jnp.minimum(off2 + cnt0, CAP - 16)
            plsc.store_scatter(pk_l, [offh + cs1 - mi1], pk1, mask=m1)
            return off2 + cnt0 + cnt1

        return lax.fori_loop(0, SCAN_CH // LANES // 2, grp2, off)

    start(0, 0)
    start(1, 1)

    def pair(p, off):
        c = p * 2
        wait(c, 0)
        off = process(c, 0, off)
        start(c + 2, 0)
        wait(c + 1, 1)
        off = process(c + 1, 1, off)
        start(c + 3, 1)
        return off

    off = lax.fori_loop(0, NCHS // 2, pair, jnp.int32(0))
    wait(NCHS, 0)
    wait(NCHS + 1, 1)
    return jnp.minimum(off, CAP - 16)


def _k4_segmax(m_in, dst, ea, bn_gamma, bn_beta):
    NEG = jnp.float32(-jnp.inf)
    AUXM = (1 << 19) - 1

    @functools.partial(
        pl.kernel, mesh=plsc.VectorSubcoreMesh(**_MESH),
        compiler_params=_SC_PARAMS,
        out_type=[jax.ShapeDtypeStruct((NPAD, 2 * CC), jnp.float32),
                  jax.ShapeDtypeStruct((NPAD, 2 * CC), jnp.float32)],
        scratch_types=[
            pltpu.VMEM((SCAN_CH,), jnp.int32),
            pltpu.VMEM((SCAN_CH,), jnp.int32),
            pltpu.VMEM((SCAN_CH,), jnp.int32),
            pltpu.VMEM((SCAN_CH,), jnp.int32),
            pltpu.SemaphoreType.DMA,
            pltpu.SemaphoreType.DMA,
            pltpu.SemaphoreType.DMA,
            pltpu.SemaphoreType.DMA,
            pltpu.VMEM((CAP,), jnp.int32),
            pltpu.VMEM((GCH,), jnp.int32),
            pltpu.VMEM((GCH,), jnp.int32),
            pltpu.VMEM((GCH, 4 * CC), jnp.float32),
            pltpu.VMEM((GCH, 4 * CC), jnp.float32),
            pltpu.VMEM((NT, 2 * CC), jnp.float32),
            pltpu.VMEM((NT, 2 * CC), jnp.float32),
            pltpu.VMEM((3, CC), jnp.float32),
            pltpu.VMEM((3, CC), jnp.float32),
            pltpu.SemaphoreType.DMA,
            pltpu.SemaphoreType.DMA,
        ],
    )
    def k(m_h, dst_h, ea_h, g_h, b_h, x12_h, x3_h, sc_d0, sc_d1, sc_e0,
          sc_e1, se0, se1, se2, se3, pk_l, eidb0, eidb1, mrows0, mrows1,
          acc01, acc2z, gam, bet, sem0, sem1):
        w = _wid()
        lo = w * NT

        ninf = jnp.full((LANES,), NEG)
        zi = jnp.zeros((LANES,), jnp.int32)

        def init_r(r, c2):
            for cg in range(2 * CC // LANES):
                sl = pl.ds(cg * LANES, LANES)
                acc01[r, sl] = ninf
                acc2z[r, sl] = ninf
            return c2

        lax.fori_loop(0, NT, init_r, jnp.int32(0))

        def init_e(r, c2):
            pk_l[pl.ds(r * LANES, LANES)] = zi
            return c2

        lax.fori_loop(0, CAP // LANES, init_e, jnp.int32(0))

        nk = _scan_compact(dst_h, ea_h, None, (sc_d0, sc_d1),
                           (sc_e0, sc_e1), None,
                           ((se0, se1), (se2, se3)), pk_l, lo, 19, 28,
                           True)
        nch = (nk + GCH - 1) // GCH
        eidbs = (eidb0, eidb1)
        mrowss = (mrows0, mrows1)
        sems = (sem0, sem1)
        ninfv = jnp.full((LANES,), NEG)

        def start_gather(j, b):
            kb = jnp.minimum(j * GCH, CAP - GCH)
            for g in range(GCH // LANES):
                pkv = pk_l[pl.ds(kb + g * LANES, LANES)]
                eidbs[b][pl.ds(g * LANES, LANES)] = pkv & AUXM
            return pltpu.async_copy(m_h.at[eidbs[b]], mrowss[b], sems[b])

        def work(j, b):
            kb = jnp.minimum(j * GCH, CAP - GCH)
            mrows = mrowss[b]

            def grp(g, c3):
                pkv = pk_l[pl.ds(kb + g * LANES, LANES)]
                for lane in range(LANES):
                    pk = pkv[lane]
                    ldst = (pk >> 19) & 511
                    ea2 = (pk >> 28) & 7
                    va = kb + g * LANES + lane < nk
                    conds = (va & (ea2 <= 2), va & (ea2 >= 2), va)
                    for kc in range(3):
                        @pl.when(conds[kc])
                        def _(kc=kc, lane=lane):
                            a = acc01 if kc < 2 else acc2z
                            ab = (kc % 2) * CC
                            for cg in range(CC // LANES):
                                sl = pl.ds(ab + cg * LANES, LANES)
                                msl = pl.ds(kc * CC + cg * LANES, LANES)
                                a[ldst, sl] = jnp.maximum(
                                    a[ldst, sl],
                                    mrows[g * LANES + lane, msl])
                return c3

            lax.fori_loop(0, GCH // LANES, grp, jnp.int32(0))

        start_gather(0, 0)
        start_gather(1, 1)
        npair = (nch + 1) // 2

        def pair(p, c2):
            j = p * 2
            pltpu.make_async_copy(m_h.at[eidb0], mrows0, sem0).wait()
            work(j, 0)
            start_gather(j + 2, 0)
            pltpu.make_async_copy(m_h.at[eidb1], mrows1, sem1).wait()
            work(j + 1, 1)
            start_gather(j + 3, 1)
            return c2

        lax.fori_loop(0, npair, pair, jnp.int32(0))
        pltpu.make_async_copy(m_h.at[eidb0], mrows0, sem0).wait()
        pltpu.make_async_copy(m_h.at[eidb1], mrows1, sem1).wait()

        pltpu.sync_copy(g_h, gam)
        pltpu.sync_copy(b_h, bet)
        zf = jnp.zeros((LANES,), jnp.float32)

        def fin_r(r, c2):
            for kc in range(3):
                a = acc01 if kc < 2 else acc2z
                cb = (kc % 2) * CC
                for cg in range(CC // LANES):
                    sl = pl.ds(cb + cg * LANES, LANES)
                    gsl = pl.ds(cg * LANES, LANES)
                    v = a[r, sl]
                    v = jnp.where(v == NEG, 0.0, v)
                    v = jnp.maximum(v * (gam[kc, gsl] * _BN_S)
                                    + bet[kc, gsl], 0.0)
                    a[r, sl] = v
            for cg in range(CC // LANES):
                acc2z[r, pl.ds(CC + cg * LANES, LANES)] = zf
            return c2

        lax.fori_loop(0, NT, fin_r, jnp.int32(0))
        pltpu.sync_copy(acc01, x12_h.at[pl.ds(lo, NT)])
        pltpu.sync_copy(acc2z, x3_h.at[pl.ds(lo, NT)])

    return k(m_in, dst, ea, bn_gamma, bn_beta)


def _k5_rgcn_sums(x12, x3, src, dst, ea):
    AUXM = (1 << 14) - 1

    @functools.partial(
        pl.kernel, mesh=plsc.VectorSubcoreMesh(**_MESH),
        compiler_params=_SC_PARAMS,
        out_type=jax.ShapeDtypeStruct((3, NPAD, 2 * CC), jnp.float32),
        scratch_types=[
            pltpu.VMEM((SCAN_CH,), jnp.int32),
            pltpu.VMEM((SCAN_CH,), jnp.int32),
            pltpu.VMEM((SCAN_CH,), jnp.int32),
            pltpu.VMEM((SCAN_CH,), jnp.int32),
            pltpu.VMEM((SCAN_CH,), jnp.int32),
            pltpu.VMEM((SCAN_CH,), jnp.int32),
            pltpu.SemaphoreType.DMA,
            pltpu.SemaphoreType.DMA,
            pltpu.SemaphoreType.DMA,
            pltpu.SemaphoreType.DMA,
            pltpu.SemaphoreType.DMA,
            pltpu.SemaphoreType.DMA,
            pltpu.VMEM((CAP,), jnp.int32),
            pltpu.VMEM((GCH,), jnp.int32),
            pltpu.VMEM((GCH,), jnp.int32),
            pltpu.VMEM((GCH, 2 * CC), jnp.float32),
            pltpu.VMEM((GCH, 2 * CC), jnp.float32),
            pltpu.VMEM((NT, 2 * CC), jnp.float32),
            pltpu.VMEM((NT, 2 * CC), jnp.float32),
            pltpu.SemaphoreType.DMA,
            pltpu.SemaphoreType.DMA,
        ],
    )
    def k(x12_h, x3_h, src_h, dst_h, ea_h, s_out, sc_d0, sc_d1, sc_e0,
          sc_e1, sc_s0, sc_s1, se0, se1, se2, se3, se4, se5, pk_l, srcb0,
          srcb1, xrows0, xrows1, accAB, accCc, sem0, sem1):
        w = _wid()
        lo = w * NT

        zi = jnp.zeros((LANES,), jnp.int32)

        def init_e(r, c2):
            pk_l[pl.ds(r * LANES, LANES)] = zi
            return c2

        lax.fori_loop(0, CAP // LANES, init_e, jnp.int32(0))

        nk = _scan_compact(dst_h, ea_h, src_h, (sc_d0, sc_d1),
                           (sc_e0, sc_e1), (sc_s0, sc_s1),
                           ((se0, se1, se2), (se3, se4, se5)), pk_l, lo,
                           14, 23, False)
        nch = (nk + GCH - 1) // GCH

        zf = jnp.zeros((LANES,), jnp.float32)
        one0 = jnp.where(lax.iota(jnp.int32, LANES) == 0, 1.0, 0.0
                         ).astype(jnp.float32)

        def zero_acc(both):
            def init_r(r, c2):
                for cg in range(2 * CC // LANES):
                    sl = pl.ds(cg * LANES, LANES)
                    accAB[r, sl] = zf
                    if both:
                        accCc[r, sl] = zf
                return c2

            lax.fori_loop(0, NT, init_r, jnp.int32(0))

        srcbs = (srcb0, srcb1)
        xrowss = (xrows0, xrows1)
        sems = (sem0, sem1)
        il = lax.iota(jnp.int32, LANES)
        cnt_vs = tuple(jnp.where(il == q, 1.0, 0.0).astype(jnp.float32)
                       for q in range(4))

        for ps in range(2):
            zero_acc(ps == 0)
            xh = x12_h if ps == 0 else x3_h

            def start_gather(j, b, xh=xh):
                kb = jnp.minimum(j * GCH, CAP - GCH)
                for g in range(GCH // LANES):
                    pkv = pk_l[pl.ds(kb + g * LANES, LANES)]
                    srcbs[b][pl.ds(g * LANES, LANES)] = pkv & AUXM
                return pltpu.async_copy(xh.at[srcbs[b]], xrowss[b], sems[b])

            def work(j, b, ps=ps):
                kb = jnp.minimum(j * GCH, CAP - GCH)
                xrows = xrowss[b]

                def grp(g, c3):
                    pkv = pk_l[pl.ds(kb + g * LANES, LANES)]
                    for lane in range(LANES):
                        pk = pkv[lane]
                        ldst = (pk >> 14) & 511
                        ea2 = (pk >> 23) & 7
                        va = kb + g * LANES + lane < nk
                        if ps == 0:
                            combos = ((va & (ea2 == 0), accAB, 0, 0, 0),
                                      (va & ((ea2 == 1) | (ea2 == 2)),
                                       accAB, CC, 0, 1),
                                      (va & (ea2 >= 2), accCc, 0, CC, 2))
                        else:
                            combos = ((va & (ea2 == 0), accAB, 0, 0, -1),
                                      (va & (ea2 > 0), accAB, CC, 0, -1))
                        for (cond, a, ab, xb, cq) in combos:
                            @pl.when(cond)
                            def _(a=a, ab=ab, xb=xb, cq=cq, lane=lane):
                                for cg in range(CC // LANES):
                                    sl = pl.ds(ab + cg * LANES, LANES)
                                    xsl = pl.ds(xb + cg * LANES, LANES)
                                    a[ldst, sl] = (a[ldst, sl]
                                                   + xrows[g * LANES + lane,
                                                           xsl])
                                if cq >= 0:
                                    csl = pl.ds(CC, LANES)
                                    accCc[ldst, csl] = (accCc[ldst, csl]
                                                        + cnt_vs[cq])

                        if ps == 0:
                            @pl.when(va)
                            def _():
                                csl = pl.ds(CC, LANES)
                                accCc[ldst, csl] = (accCc[ldst, csl]
                                                    + cnt_vs[3])
                    return c3

                lax.fori_loop(0, GCH // LANES, grp, jnp.int32(0))

            start_gather(0, 0)
            start_gather(1, 1)
            npair = (nch + 1) // 2

            def pair(p, c2, xh=xh):
                j = p * 2
                pltpu.make_async_copy(xh.at[srcb0], xrows0, sem0).wait()
                work(j, 0)
                start_gather(j + 2, 0)
                pltpu.make_async_copy(xh.at[srcb1], xrows1, sem1).wait()
                work(j + 1, 1)
                start_gather(j + 3, 1)
                return c2

            lax.fori_loop(0, npair, pair, jnp.int32(0))
            pltpu.make_async_copy(xh.at[srcb0], xrows0, sem0).wait()
            pltpu.make_async_copy(xh.at[srcb1], xrows1, sem1).wait()
            if ps == 0:
                pltpu.sync_copy(accAB, s_out.at[0, pl.ds(lo, NT)])
                pltpu.sync_copy(accCc, s_out.at[1, pl.ds(lo, NT)])
            else:
                pltpu.sync_copy(accAB, s_out.at[2, pl.ds(lo, NT)])

    return k(x12, x3, src, dst, ea)


def _k6_combine(x12, x3, s, rg_W, rg_root, rg_bias):
    RB = 1000

    def body(x12_r, x3_r, s_r, w_r, root_r, bias_r, o_r):
        x12b = x12_r[...]
        x3b = x3_r[...]
        sb = s_r[...]
        roots = root_r[...]
        ws = w_r[...]
        bias = bias_r[...]
        xs = (x12b[:, :CC], x12b[:, CC:], x3b[:, :CC])
        out = jnp.zeros((RB, CC), jnp.float32)
        for kc in range(3):
            out = out + jnp.dot(xs[kc], roots[kc],
                                preferred_element_type=jnp.float32)
            out = out + bias[kc][None, :]
        cnt0 = jnp.maximum(sb[1, :, CC:CC + 1], 1.0)
        cnt1 = jnp.maximum(sb[1, :, CC + 1:CC + 2], 1.0)
        cnt2 = jnp.maximum(sb[1, :, CC + 2:CC + 3], 1.0)
        cnt3 = jnp.maximum(sb[1, :, CC + 3:CC + 4]
                           - sb[1, :, CC:CC + 1], 1.0)
        combos = ((sb[0, :, :CC], cnt0, 0, 0),
                  (sb[0, :, CC:], cnt1, 0, 1),
                  (sb[1, :, :CC], cnt2, 1, 1),
                  (sb[2, :, :CC], cnt0, 2, 0),
                  (sb[2, :, CC:], cnt3, 2, 1))
        for (agg, cnt, kc, r) in combos:
            out = out + jnp.dot(agg / cnt, ws[kc, r],
                                preferred_element_type=jnp.float32)
        o_r[...] = out

    return pl.pallas_call(
        body,
        grid=(NN // RB,),
        in_specs=[
            pl.BlockSpec((RB, 2 * CC), lambda i: (i, 0)),
            pl.BlockSpec((RB, 2 * CC), lambda i: (i, 0)),
            pl.BlockSpec((3, RB, 2 * CC), lambda i: (0, i, 0)),
            pl.BlockSpec((3, 2, CC, CC), lambda i: (0, 0, 0, 0)),
            pl.BlockSpec((3, CC, CC), lambda i: (0, 0, 0)),
            pl.BlockSpec((3, CC), lambda i: (0, 0)),
        ],
        out_specs=pl.BlockSpec((RB, CC), lambda i: (i, 0)),
        out_shape=jax.ShapeDtypeStruct((NN, CC), jnp.float32),
    )(x12, x3, s, rg_W, rg_root, rg_bias)


def kernel(x, edge_index, edge_attr, W011, b011, gamma0, beta0, ec_W1,
           ec_b1, ec_W2, ec_b2, bn_gamma, bn_beta, rg_W, rg_root, rg_bias):
    src = edge_index[0].astype(jnp.int32)
    dst = edge_index[1].astype(jnp.int32)
    ea = edge_attr.astype(jnp.int32)

    adst, bsrc = _k1_node_tables(x, W011, b011, gamma0, beta0, ec_W1, ec_b1)
    z = _k2_edge_z(adst, bsrc, src, dst)
    m = _k3_edge_mlp(z, ec_W2, ec_b2)
    x12, x3 = _k4_segmax(m, dst, ea, bn_gamma, bn_beta)
    s = _k5_rgcn_sums(x12, x3, src, dst, ea)
    return _k6_combine(x12, x3, s, rg_W, rg_root, rg_bias)
